# Initial kernel scaffold; baseline (speedup 1.0000x reference)
import jax
import jax.numpy as jnp
from jax.experimental import pallas as pl


def kernel(x, edge_index, edge_attr, params):
    raise NotImplementedError("write your pallas kernel here")


# speedup vs baseline: 1.0039x; 1.0039x over previous
import functools
import math

import jax
import jax.numpy as jnp
from jax.experimental import pallas as pl
from jax.experimental.pallas import tpu as pltpu

_CFGS = [(8, 32, True, True), (8, 32, True, True), (1, 256, False, False)]


def _gelu(t):
    return jax.nn.gelu(t, approximate=False)


def _ln(x, g, b):
    mu = jnp.mean(x, axis=-1, keepdims=True)
    var = jnp.var(x, axis=-1, keepdims=True)
    return (x - mu) / jnp.sqrt(var + 1e-5) * g + b


def _mm_kernel(x_ref, w_ref, b_ref, o_ref):
    o_ref[...] = (
        jnp.dot(x_ref[...], w_ref[...], preferred_element_type=jnp.float32)
        + b_ref[...]
    )


def _pallas_mm(x, w, b, block_m=1000):
    m, k = x.shape
    n = w.shape[1]
    grid = (m // block_m,)
    return pl.pallas_call(
        _mm_kernel,
        grid=grid,
        in_specs=[
            pl.BlockSpec((block_m, k), lambda i: (i, 0)),
            pl.BlockSpec((k, n), lambda i: (0, 0)),
            pl.BlockSpec((n,), lambda i: (0,)),
        ],
        out_specs=pl.BlockSpec((block_m, n), lambda i: (i, 0)),
        out_shape=jax.ShapeDtypeStruct((m, n), jnp.float32),
    )(x, w, b)


def _tconv(x, src, dst, ea, p, heads, C, concat, use_beta):
    N = x.shape[0]
    q = _pallas_mm(x, p['Wq'], p['bq']).reshape(N, heads, C)
    k = _pallas_mm(x, p['Wk'], p['bk']).reshape(N, heads, C)
    v = _pallas_mm(x, p['Wv'], p['bv']).reshape(N, heads, C)
    e = (ea @ p['We']).reshape(ea.shape[0], heads, C)
    kj = k[src] + e
    qi = q[dst]
    alpha = jnp.sum(qi * kj, axis=-1) / math.sqrt(C)
    amax = jax.ops.segment_max(alpha, dst, num_segments=N)
    amax = jnp.where(jnp.isfinite(amax), amax, 0.0)
    ex = jnp.exp(alpha - amax[dst])
    den = jax.ops.segment_sum(ex, dst, num_segments=N)[dst] + 1e-16
    a = ex / den
    msg = (v[src] + e) * a[:, :, None]
    out = jax.ops.segment_sum(msg, dst, num_segments=N)
    if concat:
        out = out.reshape(N, heads * C)
    else:
        out = jnp.mean(out, axis=1)
    xr = _pallas_mm(x, p['Wskip'], p['bskip'])
    if use_beta:
        beta = jax.nn.sigmoid(jnp.concatenate([out, xr, out - xr], axis=-1) @ p['Wbeta'])
        out = beta * xr + (1.0 - beta) * out
    else:
        out = out + xr
    return out


def kernel(x, edge_index, edge_attr, params):
    src = edge_index[0]
    dst = edge_index[1]
    h = _gelu(_ln(x @ params['W_in'] + params['b_in'], params['g_in'], params['bb_in']))
    ea = edge_attr @ params['W_ep'] + params['b_ep']
    for i, (H, C, cc, ub) in enumerate(_CFGS):
        x_res = h
        keys = ['Wq', 'bq', 'Wk', 'bk', 'Wv', 'bv', 'We', 'Wskip', 'bskip']
        if ub:
            keys.append('Wbeta')
        lp = {k: params['l%d_%s' % (i, k)] for k in keys}
        h = _tconv(h, src, dst, ea, lp, H, C, cc, ub)
        h = _gelu(_ln(h, params['l%d_lng' % i], params['l%d_lnb' % i]))
        if i < len(_CFGS) - 1:
            hf = _gelu(h @ params['l%d_W1' % i] + params['l%d_b1' % i]) @ params['l%d_W2' % i] + params['l%d_b2' % i]
            h = hf + x_res
    o = _gelu(_ln(h @ params['Wo1'] + params['bo1'], params['g_o'], params['bb_o']))
    return o @ params['Wo2'] + params['bo2']


# speedup vs baseline: 2.3770x; 2.3678x over previous
import functools
import math

import jax
import jax.numpy as jnp
from jax import lax
from jax.experimental import pallas as pl
from jax.experimental.pallas import tpu as pltpu
from jax.experimental.pallas import tpu_sc as plsc

N = 10000
E = 160000
RNG = 160
NB = 64
NPAD = NB * RNG
CAP = 8192
CHE = 4000
NW = 32

_CFGS = [(8, 32, True, True), (8, 32, True, True), (1, 256, False, False)]


def _gelu(t):
    return jax.nn.gelu(t, approximate=False)


def _ln(x, g, b):
    mu = jnp.mean(x, axis=-1, keepdims=True)
    var = jnp.var(x, axis=-1, keepdims=True)
    return (x - mu) / jnp.sqrt(var + 1e-5) * g + b


def _bin_kernel(src_hbm, dst_hbm, srcs_hbm, dls_hbm, eids_hbm, cnts_hbm,
                src_ch, dst_ch, srcl0, dll0, eidl0, srcl1, dll1, eidl1,
                stage, sem):
    wid = lax.axis_index("s") * 2 + lax.axis_index("c")
    iota = lax.iota(jnp.int32, 16)
    lists = ((srcl0, dll0, eidl0), (srcl1, dll1, eidl1))

    def chunk_body(ch, cnts):
        pltpu.async_copy(src_hbm.at[pl.ds(ch * CHE, CHE)], src_ch, sem).wait()
        pltpu.async_copy(dst_hbm.at[pl.ds(ch * CHE, CHE)], dst_ch, sem).wait()

        def sub_body(sub, cnts):
            dv = dst_ch[pl.ds(sub * 16, 16)]
            sv = src_ch[pl.ds(sub * 16, 16)]
            eid = ch * CHE + sub * 16 + iota
            new = []
            for b in range(2):
                sl, dl, el = lists[b]
                cb = cnts[b]
                lo = (wid * 2 + b) * RNG
                m = (dv >= lo) & (dv < lo + RNG)
                cum = plsc.cumsum(m.astype(jnp.int32))
                pos = jnp.where(m, cb + cum - 1, CAP)
                plsc.store_scatter(sl, [pos], sv)
                plsc.store_scatter(dl, [pos], dv - lo)
                plsc.store_scatter(el, [pos], eid)
                new.append(jnp.minimum(cb + jnp.max(cum), CAP - 64))
            return tuple(new)

        return lax.fori_loop(0, CHE // 16, sub_body, cnts)

    cnts = lax.fori_loop(0, E // CHE, chunk_body,
                         (jnp.int32(0), jnp.int32(0)))

    zeros = jnp.zeros((16,), jnp.int32)
    dumm = jnp.full((16,), RNG, jnp.int32)
    for b in range(2):
        sl, dl, el = lists[b]
        cnt = cnts[b]
        for t in range(4):
            pos = cnt + t * 16 + iota
            plsc.store_scatter(sl, [pos], zeros)
            plsc.store_scatter(dl, [pos], dumm)
            plsc.store_scatter(el, [pos], zeros)
        bkt = wid * 2 + b
        pltpu.async_copy(sl.at[pl.ds(0, CAP)], srcs_hbm.at[bkt], sem).wait()
        pltpu.async_copy(dl.at[pl.ds(0, CAP)], dls_hbm.at[bkt], sem).wait()
        pltpu.async_copy(el.at[pl.ds(0, CAP)], eids_hbm.at[bkt], sem).wait()
        stage[...] = jnp.full((16,), cnt, jnp.int32)
        pltpu.async_copy(stage, cnts_hbm.at[bkt], sem).wait()


def _bin_edges(src, dst):
    mesh = plsc.VectorSubcoreMesh(core_axis_name="c", subcore_axis_name="s")
    f = functools.partial(
        pl.kernel,
        out_type=[
            jax.ShapeDtypeStruct((NB, CAP), jnp.int32),
            jax.ShapeDtypeStruct((NB, CAP), jnp.int32),
            jax.ShapeDtypeStruct((NB, CAP), jnp.int32),
            jax.ShapeDtypeStruct((NB, 16), jnp.int32),
        ],
        mesh=mesh,
        compiler_params=pltpu.CompilerParams(needs_layout_passes=False),
        scratch_types=[
            pltpu.VMEM((CHE,), jnp.int32),
            pltpu.VMEM((CHE,), jnp.int32),
            pltpu.VMEM((CAP + 16,), jnp.int32),
            pltpu.VMEM((CAP + 16,), jnp.int32),
            pltpu.VMEM((CAP + 16,), jnp.int32),
            pltpu.VMEM((CAP + 16,), jnp.int32),
            pltpu.VMEM((CAP + 16,), jnp.int32),
            pltpu.VMEM((CAP + 16,), jnp.int32),
            pltpu.VMEM((16,), jnp.int32),
            pltpu.SemaphoreType.DMA,
        ],
    )(_bin_kernel)
    return f(src, dst)


def _make_layer_kernel(H, C):
    Ww = H * 32
    HC = H * C
    mesh = plsc.VectorSubcoreMesh(core_axis_name="c", subcore_axis_name="s")

    def lk(q_hbm, k_hbm, v_hbm, qe_hbm, ea_hbm,
           srcs_hbm, dls_hbm, eids_hbm, cnts_hbm,
           zbig_hbm, zw_hbm, neg_hbm,
           vacc_hbm, wacc_hbm, den_hbm, alpha_hbm,
           qv, qe_w, ad, kv, ea_st, srcv, dlvv, eidv, alph,
           cnt_st, sem):
        wid = lax.axis_index("s") * 2 + lax.axis_index("c")
        iota = lax.iota(jnp.int32, 16)
        i32 = jnp.int32

        for rg in range(2):
            b = wid * 2 + rg
            base = b * RNG
            pltpu.sync_copy(cnts_hbm.at[b], cnt_st)
            cnt = jnp.max(cnt_st[...])
            nch = (cnt + 31) // 32

            pltpu.sync_copy(q_hbm.at[pl.ds(base, RNG)], qv.at[pl.ds(0, RNG)])
            pltpu.sync_copy(zbig_hbm.at[pl.ds(0, 8)], qv.at[pl.ds(RNG, 8)])
            pltpu.sync_copy(qe_hbm.at[pl.ds(base, RNG)], qe_w.at[pl.ds(0, RNG)])
            pltpu.sync_copy(zw_hbm.at[pl.ds(0, 8)], qe_w.at[pl.ds(RNG, 8)])
            pltpu.sync_copy(neg_hbm, ad)

            def p1_chunk(i, carry):
                pltpu.sync_copy(srcs_hbm.at[b].at[pl.ds(i * 32, 32)], srcv)
                pltpu.sync_copy(dls_hbm.at[b].at[pl.ds(i * 32, 32)], dlvv)
                pltpu.sync_copy(eids_hbm.at[b].at[pl.ds(i * 32, 32)], eidv)
                pltpu.async_copy(k_hbm.at[srcv], kv, sem).wait()
                pltpu.async_copy(ea_hbm.at[eidv], ea_st, sem).wait()

                def p1_sub(s, c2):
                    slot = s * 16 + iota
                    dlv = dlvv[pl.ds(s * 16, 16)]
                    kys, lperm = plsc.sort_key_val(dlv, iota)
                    nxt = jnp.minimum(iota + 1, 15)
                    last = (kys != kys.at[nxt].get(mode="promise_in_bounds")) \
                        | (iota == 15)
                    rowi = jnp.where(last, kys, RNG + 7)
                    for h in range(H):
                        colk = jnp.full((16,), h * C, i32)

                        def dot_c(c, acc):
                            kvals = plsc.load_gather(kv, [slot, colk + c])
                            qvals = plsc.load_gather(qv, [dlv, colk + c])
                            return acc + kvals * qvals

                        acc = lax.fori_loop(0, C, dot_c,
                                            jnp.zeros((16,), jnp.float32),
                                            unroll=8)
                        colj = jnp.full((16,), h * 32, i32)
                        colz = jnp.full((16,), 0, i32)

                        def dot_j(j, acc):
                            eav = plsc.load_gather(ea_st, [slot, colz + j])
                            qev = plsc.load_gather(qe_w, [dlv, colj + j])
                            return acc + eav * qev

                        acc = lax.fori_loop(0, 32, dot_j, acc, unroll=8)
                        alph.at[h].at[pl.ds(s * 16, 16)][...] = acc
                        a_s = acc.at[lperm].get(mode="promise_in_bounds")
                        for sh in (1, 2, 4, 8):
                            idx2 = jnp.maximum(iota - sh, 0)
                            same = kys == kys.at[idx2].get(
                                mode="promise_in_bounds")
                            cand = a_s.at[idx2].get(mode="promise_in_bounds")
                            a_s = jnp.where(same & (iota >= sh),
                                            jnp.maximum(a_s, cand), a_s)
                        colh = jnp.full((16,), h, i32)
                        cur = plsc.load_gather(ad, [kys, colh])
                        plsc.store_scatter(ad, [rowi, colh],
                                           jnp.maximum(cur, a_s))
                    return c2

                lax.fori_loop(0, 2, p1_sub, jnp.int32(0))
                pltpu.async_copy(alph, alpha_hbm.at[b].at[i], sem).wait()
                return carry

            lax.fori_loop(0, nch, p1_chunk, jnp.int32(0))

            pltpu.sync_copy(zbig_hbm, qv)
            pltpu.sync_copy(zw_hbm, qe_w)

            def p2_chunk(i, carry):
                pltpu.sync_copy(srcs_hbm.at[b].at[pl.ds(i * 32, 32)], srcv)
                pltpu.sync_copy(dls_hbm.at[b].at[pl.ds(i * 32, 32)], dlvv)
                pltpu.sync_copy(eids_hbm.at[b].at[pl.ds(i * 32, 32)], eidv)
                pltpu.async_copy(v_hbm.at[srcv], kv, sem).wait()
                pltpu.async_copy(ea_hbm.at[eidv], ea_st, sem).wait()
                pltpu.async_copy(alpha_hbm.at[b].at[i], alph, sem).wait()

                def p2_sub(s, c2):
                    slot = s * 16 + iota
                    dlv = dlvv[pl.ds(s * 16, 16)]
                    colz = jnp.full((16,), 0, i32)
                    for h in range(H):
                        ah = alph.at[h].at[pl.ds(s * 16, 16)][...]
                        colh = jnp.full((16,), h, i32)
                        mx = plsc.load_gather(ad, [dlv, colh])
                        ex = jnp.exp(ah - mx)
                        plsc.addupdate_scatter(ad, [dlv, jnp.full((16,), 16 + h, i32)], ex)
                        colj = jnp.full((16,), h * 32, i32)

                        def acc_j(j, c3):
                            eav = plsc.load_gather(ea_st, [slot, colz + j])
                            plsc.addupdate_scatter(qe_w, [dlv, colj + j],
                                                   eav * ex)
                            return c3

                        lax.fori_loop(0, 32, acc_j, jnp.int32(0), unroll=8)
                        colc = jnp.full((16,), h * C, i32)

                        def acc_c(c, c3):
                            vv = plsc.load_gather(kv, [slot, colc + c])
                            plsc.addupdate_scatter(qv, [dlv, colc + c],
                                                   vv * ex)
                            return c3

                        lax.fori_loop(0, C, acc_c, jnp.int32(0), unroll=8)
                    return c2

                lax.fori_loop(0, 2, p2_sub, jnp.int32(0))
                return carry

            lax.fori_loop(0, nch, p2_chunk, jnp.int32(0))

            pltpu.sync_copy(qv.at[pl.ds(0, RNG)], vacc_hbm.at[pl.ds(base, RNG)])
            pltpu.sync_copy(qe_w.at[pl.ds(0, RNG)],
                            wacc_hbm.at[pl.ds(base, RNG)])
            pltpu.sync_copy(ad.at[pl.ds(0, RNG)],
                            den_hbm.at[pl.ds(base, RNG)])

    f = functools.partial(
        pl.kernel,
        out_type=[
            jax.ShapeDtypeStruct((NPAD, 256), jnp.float32),
            jax.ShapeDtypeStruct((NPAD, Ww), jnp.float32),
            jax.ShapeDtypeStruct((NPAD, 32), jnp.float32),
            jax.ShapeDtypeStruct((NB, CAP // 32, 8, 32), jnp.float32),
        ],
        mesh=mesh,
        compiler_params=pltpu.CompilerParams(needs_layout_passes=False),
        scratch_types=[
            pltpu.VMEM((RNG + 8, 256), jnp.float32),
            pltpu.VMEM((RNG + 8, Ww), jnp.float32),
            pltpu.VMEM((RNG + 8, 32), jnp.float32),
            pltpu.VMEM((32, 256), jnp.float32),
            pltpu.VMEM((32, 128), jnp.float32),
            pltpu.VMEM((32,), jnp.int32),
            pltpu.VMEM((32,), jnp.int32),
            pltpu.VMEM((32,), jnp.int32),
            pltpu.VMEM((8, 32), jnp.float32),
            pltpu.VMEM((16,), jnp.int32),
            pltpu.SemaphoreType.DMA,
        ],
    )(lk)
    return f


def _graph_layer(qs, k, v, QE, ea_pad, binned, zconsts, H, C):
    srcs, dls, eids, cnts = binned
    zbig, zw_map, neg = zconsts
    f = _make_layer_kernel(H, C)
    vacc, wacc, den, _ = f(qs, k, v, QE, ea_pad, srcs, dls, eids, cnts,
                           zbig, zw_map[H * 32], neg)
    return vacc, wacc, den[:, 16:]


def _mm_kernel(x_ref, w_ref, b_ref, o_ref):
    o_ref[...] = (
        jnp.dot(x_ref[...], w_ref[...], preferred_element_type=jnp.float32)
        + b_ref[...]
    )


def _pallas_mm(x, w, b, block_m=1024):
    m, k = x.shape
    n = w.shape[1]
    grid = (m // block_m,)
    return pl.pallas_call(
        _mm_kernel,
        grid=grid,
        in_specs=[
            pl.BlockSpec((block_m, k), lambda i: (i, 0)),
            pl.BlockSpec((k, n), lambda i: (0, 0)),
            pl.BlockSpec((n,), lambda i: (0,)),
        ],
        out_specs=pl.BlockSpec((block_m, n), lambda i: (i, 0)),
        out_shape=jax.ShapeDtypeStruct((m, n), jnp.float32),
    )(x, w, b)


def _tconv_sc(h, binned, ea_pad, zconsts, p, heads, C, concat, use_beta):
    hc = heads * C
    scale = 1.0 / math.sqrt(C)
    q = (h @ p['Wq'] + p['bq']) * scale
    k = _pallas_mm(h, p['Wk'], p['bk'])
    v = _pallas_mm(h, p['Wv'], p['bv'])
    We = p['We']
    Wqe = jnp.zeros((hc, heads * 32), jnp.float32)
    Wemap = jnp.zeros((heads * 32, hc), jnp.float32)
    for hh in range(heads):
        blk = We[:, hh * C:(hh + 1) * C]
        Wqe = Wqe.at[hh * C:(hh + 1) * C, hh * 32:(hh + 1) * 32].set(blk.T)
        Wemap = Wemap.at[hh * 32:(hh + 1) * 32, hh * C:(hh + 1) * C].set(blk)
    QE = q @ Wqe
    vacc, wacc, den = _graph_layer(q, k, v, QE, ea_pad, binned, zconsts,
                                   heads, C)
    num = vacc[:, :hc] + wacc @ Wemap
    expand = jnp.repeat(jnp.eye(heads, dtype=jnp.float32), C, axis=1)
    den_exp = den[:, :heads] @ expand
    out = num / (den_exp + 1e-16)
    if not concat:
        out = jnp.mean(out.reshape(-1, heads, C), axis=1)
    xr = _pallas_mm(h, p['Wskip'], p['bskip'])
    if use_beta:
        beta = jax.nn.sigmoid(jnp.concatenate([out, xr, out - xr], axis=-1) @ p['Wbeta'])
        out = beta * xr + (1.0 - beta) * out
    else:
        out = out + xr
    return out


def kernel(x, edge_index, edge_attr, params):
    src = edge_index[0]
    dst = edge_index[1]
    binned = _bin_edges(src, dst)
    zbig = jnp.zeros((RNG + 8, 256), jnp.float32)
    zw_map = {256: zbig, 32: jnp.zeros((RNG + 8, 32), jnp.float32)}
    neg = jnp.concatenate([jnp.full((RNG + 8, 16), -3e38, jnp.float32),
                           jnp.zeros((RNG + 8, 16), jnp.float32)], axis=1)
    zconsts = (zbig, zw_map, neg)

    xpad = jnp.zeros((NPAD, x.shape[1]), jnp.float32).at[:N].set(x)
    h = _gelu(_ln(xpad @ params['W_in'] + params['b_in'],
                  params['g_in'], params['bb_in']))
    ea = edge_attr @ params['W_ep'] + params['b_ep']
    ea_pad = jnp.zeros((E, 128), jnp.float32).at[:, :32].set(ea)
    for i, (H, C, cc, ub) in enumerate(_CFGS):
        x_res = h
        keys = ['Wq', 'bq', 'Wk', 'bk', 'Wv', 'bv', 'We', 'Wskip', 'bskip']
        if ub:
            keys.append('Wbeta')
        lp = {k: params['l%d_%s' % (i, k)] for k in keys}
        h = _tconv_sc(h, binned, ea_pad, zconsts, lp, H, C, cc, ub)
        h = _gelu(_ln(h, params['l%d_lng' % i], params['l%d_lnb' % i]))
        if i < len(_CFGS) - 1:
            hf = _gelu(h @ params['l%d_W1' % i] + params['l%d_b1' % i]) @ params['l%d_W2' % i] + params['l%d_b2' % i]
            h = hf + x_res
    o = _gelu(_ln(h @ params['Wo1'] + params['bo1'], params['g_o'], params['bb_o']))
    return (o @ params['Wo2'] + params['bo2'])[:N]


# speedup vs baseline: 2.7654x; 1.1634x over previous
import functools
import math

import jax
import jax.numpy as jnp
from jax import lax
from jax.experimental import pallas as pl
from jax.experimental.pallas import tpu as pltpu
from jax.experimental.pallas import tpu_sc as plsc

N = 10000
E = 160000
RNG = 160
NB = 64
NPAD = NB * RNG
CAP = 8192
CHE = 4000
NW = 32

_CFGS = [(8, 32, True, True), (8, 32, True, True), (1, 256, False, False)]


def _gelu(t):
    return jax.nn.gelu(t, approximate=False)


def _ln(x, g, b):
    mu = jnp.mean(x, axis=-1, keepdims=True)
    var = jnp.var(x, axis=-1, keepdims=True)
    return (x - mu) / jnp.sqrt(var + 1e-5) * g + b


def _bin_kernel(src_hbm, dst_hbm, srcs_hbm, dls_hbm, eids_hbm, cnts_hbm,
                src_ch, dst_ch, srcl0, dll0, eidl0, srcl1, dll1, eidl1,
                stage, sem):
    wid = lax.axis_index("s") * 2 + lax.axis_index("c")
    iota = lax.iota(jnp.int32, 16)
    lists = ((srcl0, dll0, eidl0), (srcl1, dll1, eidl1))

    def chunk_body(ch, cnts):
        pltpu.async_copy(src_hbm.at[pl.ds(ch * CHE, CHE)], src_ch, sem).wait()
        pltpu.async_copy(dst_hbm.at[pl.ds(ch * CHE, CHE)], dst_ch, sem).wait()

        def sub_body(sub, cnts):
            dv = dst_ch[pl.ds(sub * 16, 16)]
            sv = src_ch[pl.ds(sub * 16, 16)]
            eid = ch * CHE + sub * 16 + iota
            new = []
            for b in range(2):
                sl, dl, el = lists[b]
                cb = cnts[b]
                lo = (wid * 2 + b) * RNG
                m = (dv >= lo) & (dv < lo + RNG)
                cum = plsc.cumsum(m.astype(jnp.int32))
                pos = jnp.where(m, cb + cum - 1, CAP)
                plsc.store_scatter(sl, [pos], sv)
                plsc.store_scatter(dl, [pos], dv - lo)
                plsc.store_scatter(el, [pos], eid)
                new.append(jnp.minimum(cb + jnp.max(cum), CAP - 64))
            return tuple(new)

        return lax.fori_loop(0, CHE // 16, sub_body, cnts)

    cnts = lax.fori_loop(0, E // CHE, chunk_body,
                         (jnp.int32(0), jnp.int32(0)))

    zeros = jnp.zeros((16,), jnp.int32)
    dumm = jnp.full((16,), RNG, jnp.int32)
    for b in range(2):
        sl, dl, el = lists[b]
        cnt = cnts[b]
        for t in range(4):
            pos = cnt + t * 16 + iota
            plsc.store_scatter(sl, [pos], zeros)
            plsc.store_scatter(dl, [pos], dumm)
            plsc.store_scatter(el, [pos], zeros)
        bkt = wid * 2 + b
        pltpu.async_copy(sl.at[pl.ds(0, CAP)], srcs_hbm.at[bkt], sem).wait()
        pltpu.async_copy(dl.at[pl.ds(0, CAP)], dls_hbm.at[bkt], sem).wait()
        pltpu.async_copy(el.at[pl.ds(0, CAP)], eids_hbm.at[bkt], sem).wait()
        stage[...] = jnp.full((16,), cnt, jnp.int32)
        pltpu.async_copy(stage, cnts_hbm.at[bkt], sem).wait()


def _bin_edges(src, dst):
    mesh = plsc.VectorSubcoreMesh(core_axis_name="c", subcore_axis_name="s")
    f = functools.partial(
        pl.kernel,
        out_type=[
            jax.ShapeDtypeStruct((NB, CAP), jnp.int32),
            jax.ShapeDtypeStruct((NB, CAP), jnp.int32),
            jax.ShapeDtypeStruct((NB, CAP), jnp.int32),
            jax.ShapeDtypeStruct((NB, 16), jnp.int32),
        ],
        mesh=mesh,
        compiler_params=pltpu.CompilerParams(needs_layout_passes=False),
        scratch_types=[
            pltpu.VMEM((CHE,), jnp.int32),
            pltpu.VMEM((CHE,), jnp.int32),
            pltpu.VMEM((CAP + 16,), jnp.int32),
            pltpu.VMEM((CAP + 16,), jnp.int32),
            pltpu.VMEM((CAP + 16,), jnp.int32),
            pltpu.VMEM((CAP + 16,), jnp.int32),
            pltpu.VMEM((CAP + 16,), jnp.int32),
            pltpu.VMEM((CAP + 16,), jnp.int32),
            pltpu.VMEM((16,), jnp.int32),
            pltpu.SemaphoreType.DMA,
        ],
    )(_bin_kernel)
    return f(src, dst)


def _make_layer_kernel(H, C):
    Ww = H * 32
    HC = H * C
    mesh = plsc.VectorSubcoreMesh(core_axis_name="c", subcore_axis_name="s")

    def lk(q_hbm, k_hbm, v_hbm, qe_hbm, ea_hbm,
           srcs_hbm, dls_hbm, eids_hbm, cnts_hbm,
           zbig_hbm, zw_hbm, neg_hbm,
           vacc_hbm, wacc_hbm, den_hbm, alpha_hbm,
           qv, qe_w, ad, kv, ea_st, srcv, dlvv, eidv, alph,
           cnt_st, sem):
        wid = lax.axis_index("s") * 2 + lax.axis_index("c")
        iota = lax.iota(jnp.int32, 16)
        i32 = jnp.int32

        for rg in range(2):
            b = wid * 2 + rg
            base = b * RNG
            pltpu.sync_copy(cnts_hbm.at[b], cnt_st)
            cnt = jnp.max(cnt_st[...])
            nch = (cnt + 31) // 32

            pltpu.sync_copy(q_hbm.at[pl.ds(base, RNG)], qv.at[pl.ds(0, RNG)])
            pltpu.sync_copy(zbig_hbm.at[pl.ds(0, 8)], qv.at[pl.ds(RNG, 8)])
            pltpu.sync_copy(qe_hbm.at[pl.ds(base, RNG)], qe_w.at[pl.ds(0, RNG)])
            pltpu.sync_copy(zw_hbm.at[pl.ds(0, 8)], qe_w.at[pl.ds(RNG, 8)])
            pltpu.sync_copy(neg_hbm, ad)

            def p1_chunk(i, carry):
                pltpu.sync_copy(srcs_hbm.at[b].at[pl.ds(i * 32, 32)], srcv)
                pltpu.sync_copy(dls_hbm.at[b].at[pl.ds(i * 32, 32)], dlvv)
                pltpu.sync_copy(eids_hbm.at[b].at[pl.ds(i * 32, 32)], eidv)
                pltpu.async_copy(k_hbm.at[srcv], kv, sem).wait()
                pltpu.async_copy(ea_hbm.at[eidv], ea_st, sem).wait()

                def p1_sub(s, c2):
                    slot = s * 16 + iota
                    dlv = dlvv[pl.ds(s * 16, 16)]
                    kys, lperm = plsc.sort_key_val(dlv, iota)
                    nxt = jnp.minimum(iota + 1, 15)
                    last = (kys != kys.at[nxt].get(mode="promise_in_bounds")) \
                        | (iota == 15)
                    rowi = jnp.where(last, kys, RNG + 7)
                    zv = jnp.zeros((16,), jnp.float32)
                    colz = jnp.full((16,), 0, i32)

                    def dot_j(j, accs):
                        eav = plsc.load_gather(ea_st, [slot, colz + j])
                        return tuple(
                            accs[h] + eav * plsc.load_gather(
                                qe_w, [dlv, jnp.full((16,), h * 32, i32) + j])
                            for h in range(H))

                    eaccs = lax.fori_loop(0, 32, dot_j, (zv,) * H, unroll=2)
                    for h in range(H):
                        colk = jnp.full((16,), h * C, i32)

                        def dot_c(c2, accs):
                            a0, a1, a2, a3 = accs
                            c = c2 * 4
                            a0 = a0 + plsc.load_gather(kv, [slot, colk + c]) \
                                * plsc.load_gather(qv, [dlv, colk + c])
                            a1 = a1 + plsc.load_gather(kv, [slot, colk + (c + 1)]) \
                                * plsc.load_gather(qv, [dlv, colk + (c + 1)])
                            a2 = a2 + plsc.load_gather(kv, [slot, colk + (c + 2)]) \
                                * plsc.load_gather(qv, [dlv, colk + (c + 2)])
                            a3 = a3 + plsc.load_gather(kv, [slot, colk + (c + 3)]) \
                                * plsc.load_gather(qv, [dlv, colk + (c + 3)])
                            return (a0, a1, a2, a3)

                        a0, a1, a2, a3 = lax.fori_loop(
                            0, C // 4, dot_c, (zv, zv, zv, zv), unroll=2)
                        acc = ((a0 + a1) + (a2 + a3)) + eaccs[h]
                        alph.at[h].at[pl.ds(s * 16, 16)][...] = acc
                        a_s = acc.at[lperm].get(mode="promise_in_bounds")
                        for sh in (1, 2, 4, 8):
                            idx2 = jnp.maximum(iota - sh, 0)
                            same = kys == kys.at[idx2].get(
                                mode="promise_in_bounds")
                            cand = a_s.at[idx2].get(mode="promise_in_bounds")
                            a_s = jnp.where(same & (iota >= sh),
                                            jnp.maximum(a_s, cand), a_s)
                        colh = jnp.full((16,), h, i32)
                        cur = plsc.load_gather(ad, [kys, colh])
                        plsc.store_scatter(ad, [rowi, colh],
                                           jnp.maximum(cur, a_s))
                    return c2

                lax.fori_loop(0, 2, p1_sub, jnp.int32(0))
                pltpu.async_copy(alph, alpha_hbm.at[b].at[i], sem).wait()
                return carry

            lax.fori_loop(0, nch, p1_chunk, jnp.int32(0))

            pltpu.sync_copy(zbig_hbm, qv)
            pltpu.sync_copy(zw_hbm, qe_w)

            def p2_chunk(i, carry):
                pltpu.sync_copy(srcs_hbm.at[b].at[pl.ds(i * 32, 32)], srcv)
                pltpu.sync_copy(dls_hbm.at[b].at[pl.ds(i * 32, 32)], dlvv)
                pltpu.sync_copy(eids_hbm.at[b].at[pl.ds(i * 32, 32)], eidv)
                pltpu.async_copy(v_hbm.at[srcv], kv, sem).wait()
                pltpu.async_copy(ea_hbm.at[eidv], ea_st, sem).wait()
                pltpu.async_copy(alpha_hbm.at[b].at[i], alph, sem).wait()

                def p2_sub(s, c2):
                    slot = s * 16 + iota
                    dlv = dlvv[pl.ds(s * 16, 16)]
                    colz = jnp.full((16,), 0, i32)
                    exs = []
                    for h in range(H):
                        ah = alph.at[h].at[pl.ds(s * 16, 16)][...]
                        colh = jnp.full((16,), h, i32)
                        mx = plsc.load_gather(ad, [dlv, colh])
                        ex = jnp.exp(ah - mx)
                        exs.append(ex)
                        plsc.addupdate_scatter(
                            ad, [dlv, jnp.full((16,), 16 + h, i32)], ex)

                    def acc_j(j, c3):
                        eav = plsc.load_gather(ea_st, [slot, colz + j])
                        for h in range(H):
                            plsc.addupdate_scatter(
                                qe_w, [dlv, jnp.full((16,), h * 32, i32) + j],
                                eav * exs[h])
                        return c3

                    lax.fori_loop(0, 32, acc_j, jnp.int32(0), unroll=2)
                    for h in range(H):
                        ex = exs[h]
                        colc = jnp.full((16,), h * C, i32)

                        def acc_c(c2i, c3):
                            c = c2i * 4
                            for dc in range(4):
                                vv = plsc.load_gather(kv,
                                                      [slot, colc + (c + dc)])
                                plsc.addupdate_scatter(qv,
                                                       [dlv, colc + (c + dc)],
                                                       vv * ex)
                            return c3

                        lax.fori_loop(0, C // 4, acc_c, jnp.int32(0),
                                      unroll=2)
                    return c2

                lax.fori_loop(0, 2, p2_sub, jnp.int32(0))
                return carry

            lax.fori_loop(0, nch, p2_chunk, jnp.int32(0))

            pltpu.sync_copy(qv.at[pl.ds(0, RNG)], vacc_hbm.at[pl.ds(base, RNG)])
            pltpu.sync_copy(qe_w.at[pl.ds(0, RNG)],
                            wacc_hbm.at[pl.ds(base, RNG)])
            pltpu.sync_copy(ad.at[pl.ds(0, RNG)],
                            den_hbm.at[pl.ds(base, RNG)])

    f = functools.partial(
        pl.kernel,
        out_type=[
            jax.ShapeDtypeStruct((NPAD, 256), jnp.float32),
            jax.ShapeDtypeStruct((NPAD, Ww), jnp.float32),
            jax.ShapeDtypeStruct((NPAD, 32), jnp.float32),
            jax.ShapeDtypeStruct((NB, CAP // 32, 8, 32), jnp.float32),
        ],
        mesh=mesh,
        compiler_params=pltpu.CompilerParams(needs_layout_passes=False),
        scratch_types=[
            pltpu.VMEM((RNG + 8, 256), jnp.float32),
            pltpu.VMEM((RNG + 8, Ww), jnp.float32),
            pltpu.VMEM((RNG + 8, 32), jnp.float32),
            pltpu.VMEM((32, 256), jnp.float32),
            pltpu.VMEM((32, 128), jnp.float32),
            pltpu.VMEM((32,), jnp.int32),
            pltpu.VMEM((32,), jnp.int32),
            pltpu.VMEM((32,), jnp.int32),
            pltpu.VMEM((8, 32), jnp.float32),
            pltpu.VMEM((16,), jnp.int32),
            pltpu.SemaphoreType.DMA,
        ],
    )(lk)
    return f


def _graph_layer(qs, k, v, QE, ea_pad, binned, zconsts, H, C):
    srcs, dls, eids, cnts = binned
    zbig, zw_map, neg = zconsts
    f = _make_layer_kernel(H, C)
    vacc, wacc, den, _ = f(qs, k, v, QE, ea_pad, srcs, dls, eids, cnts,
                           zbig, zw_map[H * 32], neg)
    return vacc, wacc, den[:, 16:]


def _mm_kernel(x_ref, w_ref, b_ref, o_ref):
    o_ref[...] = (
        jnp.dot(x_ref[...], w_ref[...], preferred_element_type=jnp.float32)
        + b_ref[...]
    )


def _pallas_mm(x, w, b, block_m=1024):
    m, k = x.shape
    n = w.shape[1]
    grid = (m // block_m,)
    return pl.pallas_call(
        _mm_kernel,
        grid=grid,
        in_specs=[
            pl.BlockSpec((block_m, k), lambda i: (i, 0)),
            pl.BlockSpec((k, n), lambda i: (0, 0)),
            pl.BlockSpec((n,), lambda i: (0,)),
        ],
        out_specs=pl.BlockSpec((block_m, n), lambda i: (i, 0)),
        out_shape=jax.ShapeDtypeStruct((m, n), jnp.float32),
    )(x, w, b)


def _tconv_sc(h, binned, ea_pad, zconsts, p, heads, C, concat, use_beta):
    hc = heads * C
    scale = 1.0 / math.sqrt(C)
    q = _pallas_mm(h, p['Wq'] * scale, p['bq'] * scale)
    k = _pallas_mm(h, p['Wk'], p['bk'])
    v = _pallas_mm(h, p['Wv'], p['bv'])
    We = p['We']
    Wqe = jnp.zeros((hc, heads * 32), jnp.float32)
    Wemap = jnp.zeros((heads * 32, hc), jnp.float32)
    for hh in range(heads):
        blk = We[:, hh * C:(hh + 1) * C]
        Wqe = Wqe.at[hh * C:(hh + 1) * C, hh * 32:(hh + 1) * 32].set(blk.T)
        Wemap = Wemap.at[hh * 32:(hh + 1) * 32, hh * C:(hh + 1) * C].set(blk)
    QE = _pallas_mm(q, Wqe, jnp.zeros((heads * 32,), jnp.float32))
    vacc, wacc, den = _graph_layer(q, k, v, QE, ea_pad, binned, zconsts,
                                   heads, C)
    num = vacc[:, :hc] + _pallas_mm(wacc, Wemap, jnp.zeros((hc,), jnp.float32))
    expand = jnp.repeat(jnp.eye(heads, dtype=jnp.float32), C, axis=1)
    den_exp = den[:, :heads] @ expand
    out = num / (den_exp + 1e-16)
    if not concat:
        out = jnp.mean(out.reshape(-1, heads, C), axis=1)
    xr = _pallas_mm(h, p['Wskip'], p['bskip'])
    if use_beta:
        beta = jax.nn.sigmoid(jnp.concatenate([out, xr, out - xr], axis=-1) @ p['Wbeta'])
        out = beta * xr + (1.0 - beta) * out
    else:
        out = out + xr
    return out


def kernel(x, edge_index, edge_attr, params):
    src = edge_index[0]
    dst = edge_index[1]
    binned = _bin_edges(src, dst)
    zbig = jnp.zeros((RNG + 8, 256), jnp.float32)
    zw_map = {256: zbig, 32: jnp.zeros((RNG + 8, 32), jnp.float32)}
    neg = jnp.concatenate([jnp.full((RNG + 8, 16), -3e38, jnp.float32),
                           jnp.zeros((RNG + 8, 16), jnp.float32)], axis=1)
    zconsts = (zbig, zw_map, neg)

    xpad = jnp.zeros((NPAD, x.shape[1]), jnp.float32).at[:N].set(x)
    h = _gelu(_ln(_pallas_mm(xpad, params['W_in'], params['b_in']),
                  params['g_in'], params['bb_in']))
    ea = _pallas_mm(edge_attr, params['W_ep'], params['b_ep'], block_m=1000)
    ea_pad = jnp.zeros((E, 128), jnp.float32).at[:, :32].set(ea)
    for i, (H, C, cc, ub) in enumerate(_CFGS):
        x_res = h
        keys = ['Wq', 'bq', 'Wk', 'bk', 'Wv', 'bv', 'We', 'Wskip', 'bskip']
        if ub:
            keys.append('Wbeta')
        lp = {k: params['l%d_%s' % (i, k)] for k in keys}
        h = _tconv_sc(h, binned, ea_pad, zconsts, lp, H, C, cc, ub)
        h = _gelu(_ln(h, params['l%d_lng' % i], params['l%d_lnb' % i]))
        if i < len(_CFGS) - 1:
            h1 = _gelu(_pallas_mm(h, params['l%d_W1' % i], params['l%d_b1' % i]))
            hf = _pallas_mm(h1, params['l%d_W2' % i], params['l%d_b2' % i])
            h = hf + x_res
    o = _gelu(_ln(_pallas_mm(h, params['Wo1'], params['bo1']),
                  params['g_o'], params['bb_o']))
    return (o @ params['Wo2'] + params['bo2'])[:N]
